# baseline (speedup 1.0000x reference)
import functools

import jax
import jax.numpy as jnp
from jax import lax
from jax.experimental import pallas as pl
from jax.experimental.pallas import tpu as pltpu
from jax.experimental.pallas import tpu_sc as plsc


def _make_sc_lookup(B, V, D):
    info = plsc.get_sparse_core_info()
    NC, NS = info.num_cores, info.num_subcores
    NW = NC * NS
    assert B % NW == 0
    b_per_w = B // NW
    CH = 64
    assert b_per_w % CH == 0
    n_ch = b_per_w // CH

    mesh = plsc.VectorSubcoreMesh(core_axis_name="c", subcore_axis_name="s")

    @functools.partial(
        pl.kernel,
        out_type=jax.ShapeDtypeStruct((B, D), jnp.float32),
        mesh=mesh,
        scratch_types=[
            pltpu.VMEM((b_per_w,), jnp.int32),
            pltpu.VMEM((b_per_w, D), jnp.float32),
            pltpu.SemaphoreType.DMA((n_ch,)),
            pltpu.SemaphoreType.DMA((n_ch,)),
            pltpu.SemaphoreType.DMA,
        ],
    )
    def lookup(labels_hbm, table_hbm, out_hbm, idx_v, rows_v, lsem, gsem, wsem):
        wid = lax.axis_index("s") * NC + lax.axis_index("c")
        base = wid * b_per_w
        loads = []
        for j in range(n_ch):
            loads.append(
                pltpu.async_copy(
                    labels_hbm.at[pl.ds(base + j * CH, CH)],
                    idx_v.at[pl.ds(j * CH, CH)],
                    lsem.at[j],
                )
            )
        gathers = []
        for j in range(n_ch):
            loads[j].wait()
            gathers.append(
                pltpu.async_copy(
                    table_hbm.at[idx_v.at[pl.ds(j * CH, CH)]],
                    rows_v.at[pl.ds(j * CH, CH)],
                    gsem.at[j],
                )
            )
        writes = []
        for j in range(n_ch):
            gathers[j].wait()
            writes.append(
                pltpu.async_copy(
                    rows_v.at[pl.ds(j * CH, CH)],
                    out_hbm.at[pl.ds(base + j * CH, CH)],
                    wsem,
                )
            )
        for w in writes:
            w.wait()

    return lookup


def kernel(labels, table):
    B, = labels.shape
    V, D = table.shape
    lookup = _make_sc_lookup(B, V, D)
    return lookup(labels.astype(jnp.int32), table)


# speedup vs baseline: 1.0318x; 1.0318x over previous
import functools

import jax
import jax.numpy as jnp
from jax import lax
from jax.experimental import pallas as pl
from jax.experimental.pallas import tpu as pltpu
from jax.experimental.pallas import tpu_sc as plsc


def _make_sc_lookup(B, V, D):
    info = plsc.get_sparse_core_info()
    NC, NS = info.num_cores, info.num_subcores
    NW = NC * NS
    assert B % NW == 0
    b_per_w = B // NW
    CH = 512
    assert b_per_w % CH == 0
    n_ch = b_per_w // CH

    mesh = plsc.VectorSubcoreMesh(core_axis_name="c", subcore_axis_name="s")

    @functools.partial(
        pl.kernel,
        out_type=jax.ShapeDtypeStruct((B, D), jnp.float32),
        mesh=mesh,
        scratch_types=[
            pltpu.VMEM((b_per_w,), jnp.int32),
            pltpu.VMEM((b_per_w, D), jnp.float32),
            pltpu.SemaphoreType.DMA((n_ch,)),
            pltpu.SemaphoreType.DMA((n_ch,)),
            pltpu.SemaphoreType.DMA,
        ],
    )
    def lookup(labels_hbm, table_hbm, out_hbm, idx_v, rows_v, lsem, gsem, wsem):
        wid = lax.axis_index("c") * NS + lax.axis_index("s")
        base = wid * b_per_w
        loads = []
        for j in range(n_ch):
            loads.append(
                pltpu.async_copy(
                    labels_hbm.at[pl.ds(base + j * CH, CH)],
                    idx_v.at[pl.ds(j * CH, CH)],
                    lsem.at[j],
                )
            )
        gathers = []
        for j in range(n_ch):
            loads[j].wait()
            gathers.append(
                pltpu.async_copy(
                    table_hbm.at[idx_v.at[pl.ds(j * CH, CH)]],
                    rows_v.at[pl.ds(j * CH, CH)],
                    gsem.at[j],
                )
            )
        writes = []
        for j in range(n_ch):
            gathers[j].wait()
            writes.append(
                pltpu.async_copy(
                    rows_v.at[pl.ds(j * CH, CH)],
                    out_hbm.at[pl.ds(base + j * CH, CH)],
                    wsem,
                )
            )
        for w in writes:
            w.wait()

    return lookup


def kernel(labels, table):
    B, = labels.shape
    V, D = table.shape
    lookup = _make_sc_lookup(B, V, D)
    return lookup(labels.astype(jnp.int32), table)
